# baseline (speedup 1.0000x reference)
import functools

import jax
import jax.numpy as jnp
from jax import lax
from jax.experimental import pallas as pl
from jax.experimental.pallas import tpu as pltpu
from jax.experimental.pallas import tpu_sc as plsc

N = 10000
NP = 12288
H = 128
CH = 128
NTILES = 32
OWN = NP // NTILES
ACC_ROWS = OWN + 8
_f32 = jnp.float32

_mesh = plsc.VectorSubcoreMesh(core_axis_name="c", subcore_axis_name="s")
SUP = 8192
SUPCH = SUP // CH


def _dyn_lane(vec, lane):
    idx = jnp.full((16,), lane, jnp.int32)
    return vec.at[idx].get(mode="promise_in_bounds")[0]


def _acc_rows_chunk(acc, rows_v, lidx_v):
    def body(q, carry):
        base = pl.multiple_of(q * 16, 16)
        rvec = lidx_v[pl.ds(base, 16)]
        for l in range(16):
            r = rvec[l]
            e = base + l
            for j in range(H // 16):
                sl = pl.ds(j * 16, 16)
                acc[r, sl] = acc[r, sl] + rows_v[e, sl]
        return carry

    lax.fori_loop(0, CH // 16, body, 0)


@functools.partial(
    pl.kernel,
    out_type=[
        jax.ShapeDtypeStruct((NP, H), _f32),
        jax.ShapeDtypeStruct((NP, 16), _f32),
    ],
    mesh=_mesh,
    scratch_types=[
        pltpu.VMEM((CH,), jnp.int32),
        pltpu.VMEM((CH,), jnp.int32),
        pltpu.VMEM((CH, H), _f32),
        pltpu.VMEM((CH, H), _f32),
        pltpu.VMEM((ACC_ROWS, 16), _f32),
        pltpu.VMEM((CH + 16,), jnp.int32),
        pltpu.VMEM((NTILES, 16), jnp.int32),
        pltpu.SemaphoreType.DMA,
    ],
)
def _sc_embed_deg(xa_h, xb_h, cls_h, nbr_h, pidx_h, offs_h,
                  h0_h, degp_h,
                  idxa_v, idxb_v, rows_v, rows2_v, dacc, exidx_v,
                  offs_v, sem):
    c = lax.axis_index("c")
    s = lax.axis_index("s")
    g = c * 16 + s

    for k in range(OWN // CH):
        base = g * OWN + k * CH
        pltpu.sync_copy(xa_h.at[pl.ds(base, CH)], idxa_v)
        pltpu.sync_copy(xb_h.at[pl.ds(base, CH)], idxb_v)
        pltpu.async_copy(cls_h.at[idxa_v], rows_v, sem).wait()
        pltpu.async_copy(nbr_h.at[idxb_v], rows2_v, sem).wait()

        def addrow(r, carry):
            for j in range(H // 16):
                sl = pl.ds(j * 16, 16)
                rows_v[r, sl] = rows_v[r, sl] + rows2_v[r, sl]
            return carry

        lax.fori_loop(0, CH, addrow, 0)
        pltpu.sync_copy(rows_v, h0_h.at[pl.ds(base, CH)])

    def zrow(r, carry):
        dacc[r, pl.ds(0, 16)] = jnp.zeros((16,), _f32)
        return carry

    lax.fori_loop(0, ACC_ROWS, zrow, 0)
    pltpu.sync_copy(offs_h, offs_v)
    ovec = offs_v[g, pl.ds(0, 16)]
    o_g = pl.multiple_of(ovec[0], CH)
    n_g = ovec[1]

    def dchunk(i, carry):
        off = o_g + i * CH
        pltpu.sync_copy(pidx_h.at[pl.ds(off, CH)],
                        exidx_v.at[pl.ds(0, CH)])

        def dedge(q, carry2):
            base = pl.multiple_of(q * 16, 16)
            rvec = exidx_v[pl.ds(base, 16)] & 511
            for l in range(16):
                r = rvec[l]
                dacc[r, pl.ds(0, 16)] = dacc[r, pl.ds(0, 16)] + 1.0
            return carry2

        lax.fori_loop(0, CH // 16, dedge, 0)
        return carry

    lax.fori_loop(0, n_g, dchunk, 0)

    for k in range(OWN // CH):
        pltpu.sync_copy(dacc.at[pl.ds(k * CH, CH)],
                        degp_h.at[pl.ds(g * OWN + k * CH, CH)])


@functools.partial(
    pl.kernel,
    out_type=jax.ShapeDtypeStruct((NP, H), _f32),
    mesh=_mesh,
    scratch_types=[
        pltpu.VMEM((SUP,), jnp.int32),
        pltpu.VMEM((CH,), jnp.int32),
        pltpu.VMEM((CH,), jnp.int32),
        pltpu.VMEM((CH, H), _f32),
        pltpu.VMEM((CH, H), _f32),
        pltpu.VMEM((ACC_ROWS, H), _f32),
        pltpu.VMEM((NTILES, 16), jnp.int32),
        pltpu.SemaphoreType.DMA,
        pltpu.SemaphoreType.DMA,
    ],
)
def _sc_agg(hs_h, pidx_h, offs_h,
            p_h,
            bigidx_v, src_a, src_b, rows_a, rows_b, acc, offs_v,
            sem_a, sem_b):
    c = lax.axis_index("c")
    s = lax.axis_index("s")
    g = c * 16 + s

    for k in range(OWN // CH):
        pltpu.sync_copy(hs_h.at[pl.ds(g * OWN + k * CH, CH)],
                        acc.at[pl.ds(k * CH, CH)])
    pltpu.sync_copy(offs_h, offs_v)
    ovec = offs_v[g, pl.ds(0, 16)]
    o_g = pl.multiple_of(ovec[0], CH)
    n_g = ovec[1]
    n_sup = (n_g + SUPCH - 1) // SUPCH

    def _unpack(i, sbuf):
        cb = i * CH
        for q in range(CH // 16):
            v = bigidx_v[pl.ds(cb + q * 16, 16)]
            sbuf[pl.ds(q * 16, 16)] = lax.shift_right_logical(v, 9)

    def _accum(i, rbuf):
        cb = i * CH
        for q in range(CH // 16):
            rvec = bigidx_v[pl.ds(cb + q * 16, 16)] & 511
            for l in range(16):
                r = rvec[l]
                e = q * 16 + l
                for j in range(H // 16):
                    sl = pl.ds(j * 16, 16)
                    plsc.addupdate(acc.at[r, sl], rbuf[e, sl])

    def sup_body(sb, carry):
        off = pl.multiple_of(o_g + sb * SUP, CH)
        pltpu.sync_copy(pidx_h.at[pl.ds(off, SUP)], bigidx_v)
        nch = jnp.minimum(n_g - sb * SUPCH, SUPCH)

        _unpack(0, src_a)
        pltpu.async_copy(hs_h.at[src_a], rows_a, sem_a)

        def pair(p, carry2):
            i0 = 2 * p
            i1 = i0 + 1

            @pl.when(i1 < nch)
            def _():
                _unpack(i1, src_b)
                pltpu.async_copy(hs_h.at[src_b], rows_b, sem_b)

            pltpu.make_async_copy(hs_h.at[src_a], rows_a, sem_a).wait()
            _accum(i0, rows_a)

            @pl.when(i0 + 2 < nch)
            def _():
                _unpack(i0 + 2, src_a)
                pltpu.async_copy(hs_h.at[src_a], rows_a, sem_a)

            @pl.when(i1 < nch)
            def _():
                pltpu.make_async_copy(hs_h.at[src_b], rows_b, sem_b).wait()
                _accum(i1, rows_b)

            return carry2

        lax.fori_loop(0, (nch + 1) // 2, pair, 0)
        return carry

    lax.fori_loop(0, n_sup, sup_body, 0)

    for k in range(OWN // CH):
        pltpu.sync_copy(acc.at[pl.ds(k * CH, CH)],
                        p_h.at[pl.ds(g * OWN + k * CH, CH)])


def _matT(a, w):
    return lax.dot_general(a, w, (((1,), (1,)), ((), ())),
                           preferred_element_type=_f32)


def _tc_prep_body(degp_ref, h0_ref, w_ref, dinv_ref, hs_ref):
    deg = degp_ref[:, 0:1] + 1.0
    dinv = lax.rsqrt(deg)
    dinv_ref[...] = dinv
    hs_ref[...] = _matT(h0_ref[...], w_ref[...]) * dinv


def _layer_tail(p, h, dinv, b, gam, bet):
    out = p * dinv + b
    hn = h + jnp.maximum(out, 0.0)
    mu = jnp.mean(hn, axis=1, keepdims=True)
    d = hn - mu
    var = jnp.mean(d * d, axis=1, keepdims=True)
    return d * lax.rsqrt(var + 1e-5) * gam + bet


def _tc_mid_body(p_ref, h_ref, dinv_ref, b_ref, g_ref, be_ref,
                 w_ref, hn_ref, hs_ref):
    dinv = dinv_ref[...]
    hln = _layer_tail(p_ref[...], h_ref[...], dinv,
                      b_ref[...], g_ref[...], be_ref[...])
    hn_ref[...] = hln
    hs_ref[...] = _matT(hln, w_ref[...]) * dinv


def _tc_fin_body(p_ref, h_ref, dinv_ref, b_ref, g_ref, be_ref,
                 root_ref, wout_ref, out_ref, h3_ref):
    hln = _layer_tail(p_ref[...], h_ref[...], dinv_ref[...],
                      b_ref[...], g_ref[...], be_ref[...])
    h3_ref[...] = hln
    rows = [h3_ref[pl.ds(root_ref[i], 1), :] for i in range(10)]
    rows.append(jnp.zeros((6, H), _f32))
    r = jnp.concatenate(rows, axis=0)
    out_ref[...] = _matT(r, wout_ref[...])


def _tc_prep(degp, h0, w0):
    return pl.pallas_call(
        _tc_prep_body,
        out_shape=[jax.ShapeDtypeStruct((NP, 1), _f32),
                   jax.ShapeDtypeStruct((NP, H), _f32)],
    )(degp, h0, w0)


def _tc_mid(p, h, dinv, b, gam, bet, w_next):
    return pl.pallas_call(
        _tc_mid_body,
        out_shape=[jax.ShapeDtypeStruct((NP, H), _f32),
                   jax.ShapeDtypeStruct((NP, H), _f32)],
    )(p, h, dinv, b, gam, bet, w_next)


def _tc_fin(p, h, dinv, b, gam, bet, root16, w_out):
    vm = pl.BlockSpec(memory_space=pltpu.VMEM)
    return pl.pallas_call(
        _tc_fin_body,
        out_shape=jax.ShapeDtypeStruct((16, 64), _f32),
        in_specs=[vm] * 6 + [pl.BlockSpec(memory_space=pltpu.SMEM), vm],
        scratch_shapes=[pltpu.VMEM((NP, H), _f32)],
    )(p, h, dinv, b, gam, bet, root16, w_out)


def kernel(x, edge_index, batch, cls_emb, nbr_emb, Ws, bs, gammas, betas,
           W_out):
    E = edge_index.shape[1]
    EPS = E + NTILES * CH + SUP

    src_e = edge_index[0].astype(jnp.int32)
    dst_e = edge_index[1].astype(jnp.int32)

    own = dst_e // OWN
    nblk = E // CH
    oh = (own.reshape(nblk, CH)[None, :, :]
          == jnp.arange(NTILES, dtype=jnp.int32)[:, None, None])
    intra = jnp.cumsum(oh.astype(jnp.int16), axis=2)
    blksum = intra[:, :, -1].astype(jnp.int32)
    blkoff = jnp.cumsum(blksum, axis=1) - blksum
    cnt = blkoff[:, -1] + blksum[:, -1]
    pad = ((cnt + CH - 1) // CH) * CH
    offs = jnp.concatenate([jnp.zeros((1,), jnp.int32),
                            jnp.cumsum(pad, dtype=jnp.int32)[:-1]])
    eidx = jnp.arange(E, dtype=jnp.int32)
    rank = (intra[own, eidx // CH, eidx % CH].astype(jnp.int32) - 1
            + blkoff[own, eidx // CH])
    slot = offs[own] + rank
    packed = jnp.full((EPS,), OWN, jnp.int32).at[slot].set(
        (src_e << 9) | (dst_e % OWN))
    offs2d = (jnp.zeros((NTILES, 16), jnp.int32)
              .at[:, 0].set(offs).at[:, 1].set(pad // CH))

    xa = jnp.pad(x[:, 0].astype(jnp.int32), (0, NP - N))
    xb = jnp.pad(x[:, 1].astype(jnp.int32), (0, NP - N))
    roots = jnp.searchsorted(batch.astype(jnp.int32),
                             jnp.arange(10, dtype=jnp.int32)).astype(jnp.int32)
    root16 = jnp.pad(roots, (0, 6))

    h0, degp = _sc_embed_deg(xa, xb, cls_emb.astype(_f32),
                             nbr_emb.astype(_f32), packed, offs2d)
    dinv, hs = _tc_prep(degp, h0, Ws[0])

    h = h0
    out16 = None
    for l in range(Ws.shape[0]):
        p = _sc_agg(hs, packed, offs2d)
        b = bs[l].reshape(1, H)
        gam = gammas[l].reshape(1, H)
        bet = betas[l].reshape(1, H)
        if l + 1 < Ws.shape[0]:
            h, hs = _tc_mid(p, h, dinv, b, gam, bet, Ws[l + 1])
        else:
            out16 = _tc_fin(p, h, dinv, b, gam, bet, root16, W_out)
    return out16[:10]


# speedup vs baseline: 2.3674x; 2.3674x over previous
import functools

import jax
import jax.numpy as jnp
from jax import lax
from jax.experimental import pallas as pl
from jax.experimental.pallas import tpu as pltpu
from jax.experimental.pallas import tpu_sc as plsc

N = 10000
NP = 12288
H = 128
CH = 128
NTILES = 32
OWN = NP // NTILES
ACC_ROWS = OWN + 8
_f32 = jnp.float32

_mesh = plsc.VectorSubcoreMesh(core_axis_name="c", subcore_axis_name="s")
SUP = 8192
SUPCH = SUP // CH


def _dyn_lane(vec, lane):
    idx = jnp.full((16,), lane, jnp.int32)
    return vec.at[idx].get(mode="promise_in_bounds")[0]


def _acc_rows_chunk(acc, rows_v, lidx_v):
    def body(q, carry):
        base = pl.multiple_of(q * 16, 16)
        rvec = lidx_v[pl.ds(base, 16)]
        for l in range(16):
            r = rvec[l]
            e = base + l
            for j in range(H // 16):
                sl = pl.ds(j * 16, 16)
                acc[r, sl] = acc[r, sl] + rows_v[e, sl]
        return carry

    lax.fori_loop(0, CH // 16, body, 0)


@functools.partial(
    pl.kernel,
    out_type=[
        jax.ShapeDtypeStruct((NP, H), _f32),
        jax.ShapeDtypeStruct((NP, 16), _f32),
    ],
    mesh=_mesh,
    scratch_types=[
        pltpu.VMEM((CH,), jnp.int32),
        pltpu.VMEM((CH,), jnp.int32),
        pltpu.VMEM((CH, H), _f32),
        pltpu.VMEM((CH, H), _f32),
        pltpu.VMEM((ACC_ROWS, 16), _f32),
        pltpu.VMEM((CH + 16,), jnp.int32),
        pltpu.VMEM((NTILES, 16), jnp.int32),
        pltpu.SemaphoreType.DMA,
    ],
)
def _sc_embed_deg(xa_h, xb_h, cls_h, nbr_h, pidx_h, offs_h,
                  h0_h, degp_h,
                  idxa_v, idxb_v, rows_v, rows2_v, dacc, exidx_v,
                  offs_v, sem):
    c = lax.axis_index("c")
    s = lax.axis_index("s")
    g = c * 16 + s

    for k in range(OWN // CH):
        base = g * OWN + k * CH
        pltpu.sync_copy(xa_h.at[pl.ds(base, CH)], idxa_v)
        pltpu.sync_copy(xb_h.at[pl.ds(base, CH)], idxb_v)
        pltpu.async_copy(cls_h.at[idxa_v], rows_v, sem).wait()
        pltpu.async_copy(nbr_h.at[idxb_v], rows2_v, sem).wait()

        def addrow(r, carry):
            for j in range(H // 16):
                sl = pl.ds(j * 16, 16)
                rows_v[r, sl] = rows_v[r, sl] + rows2_v[r, sl]
            return carry

        lax.fori_loop(0, CH, addrow, 0)
        pltpu.sync_copy(rows_v, h0_h.at[pl.ds(base, CH)])

    def zrow(r, carry):
        dacc[r, pl.ds(0, 16)] = jnp.zeros((16,), _f32)
        return carry

    lax.fori_loop(0, ACC_ROWS, zrow, 0)
    pltpu.sync_copy(offs_h, offs_v)
    ovec = offs_v[g, pl.ds(0, 16)]
    o_g = pl.multiple_of(ovec[0], CH)
    n_g = ovec[1]

    def dchunk(i, carry):
        off = o_g + i * CH
        pltpu.sync_copy(pidx_h.at[pl.ds(off, CH)],
                        exidx_v.at[pl.ds(0, CH)])

        def dedge(q, carry2):
            base = pl.multiple_of(q * 16, 16)
            rvec = exidx_v[pl.ds(base, 16)] & 511
            for l in range(16):
                r = rvec[l]
                dacc[r, pl.ds(0, 16)] = dacc[r, pl.ds(0, 16)] + 1.0
            return carry2

        lax.fori_loop(0, CH // 16, dedge, 0)
        return carry

    lax.fori_loop(0, n_g, dchunk, 0)

    for k in range(OWN // CH):
        pltpu.sync_copy(dacc.at[pl.ds(k * CH, CH)],
                        degp_h.at[pl.ds(g * OWN + k * CH, CH)])


@functools.partial(
    pl.kernel,
    out_type=jax.ShapeDtypeStruct((NP, H), _f32),
    mesh=_mesh,
    scratch_types=[
        pltpu.VMEM((SUP,), jnp.int32),
        pltpu.VMEM((CH,), jnp.int32),
        pltpu.VMEM((CH,), jnp.int32),
        pltpu.VMEM((CH, H), _f32),
        pltpu.VMEM((CH, H), _f32),
        pltpu.VMEM((ACC_ROWS, H), _f32),
        pltpu.VMEM((NTILES, 16), jnp.int32),
        pltpu.SemaphoreType.DMA,
        pltpu.SemaphoreType.DMA,
    ],
)
def _sc_agg(hs_h, pidx_h, offs_h,
            p_h,
            bigidx_v, src_a, src_b, rows_a, rows_b, acc, offs_v,
            sem_a, sem_b):
    c = lax.axis_index("c")
    s = lax.axis_index("s")
    g = c * 16 + s

    for k in range(OWN // CH):
        pltpu.sync_copy(hs_h.at[pl.ds(g * OWN + k * CH, CH)],
                        acc.at[pl.ds(k * CH, CH)])
    pltpu.sync_copy(offs_h, offs_v)
    ovec = offs_v[g, pl.ds(0, 16)]
    o_g = pl.multiple_of(ovec[0], CH)
    n_g = ovec[1]
    n_sup = (n_g + SUPCH - 1) // SUPCH

    def _unpack(i, sbuf):
        cb = i * CH
        for q in range(CH // 16):
            v = bigidx_v[pl.ds(cb + q * 16, 16)]
            sbuf[pl.ds(q * 16, 16)] = lax.shift_right_logical(v, 9)

    def _accum(i, rbuf):
        cb = i * CH
        for q in range(CH // 16):
            rvec = bigidx_v[pl.ds(cb + q * 16, 16)] & 511
            for l in range(16):
                r = rvec[l]
                e = q * 16 + l
                for j in range(H // 16):
                    sl = pl.ds(j * 16, 16)
                    plsc.addupdate(acc.at[r, sl], rbuf[e, sl])

    def sup_body(sb, carry):
        off = pl.multiple_of(o_g + sb * SUP, CH)
        pltpu.sync_copy(pidx_h.at[pl.ds(off, SUP)], bigidx_v)
        nch = jnp.minimum(n_g - sb * SUPCH, SUPCH)

        _unpack(0, src_a)
        pltpu.async_copy(hs_h.at[src_a], rows_a, sem_a)

        def pair(p, carry2):
            i0 = 2 * p
            i1 = i0 + 1

            @pl.when(i1 < nch)
            def _():
                _unpack(i1, src_b)
                pltpu.async_copy(hs_h.at[src_b], rows_b, sem_b)

            pltpu.make_async_copy(hs_h.at[src_a], rows_a, sem_a).wait()
            _accum(i0, rows_a)

            @pl.when(i0 + 2 < nch)
            def _():
                _unpack(i0 + 2, src_a)
                pltpu.async_copy(hs_h.at[src_a], rows_a, sem_a)

            @pl.when(i1 < nch)
            def _():
                pltpu.make_async_copy(hs_h.at[src_b], rows_b, sem_b).wait()
                _accum(i1, rows_b)

            return carry2

        lax.fori_loop(0, (nch + 1) // 2, pair, 0)
        return carry

    lax.fori_loop(0, n_sup, sup_body, 0)

    for k in range(OWN // CH):
        pltpu.sync_copy(acc.at[pl.ds(k * CH, CH)],
                        p_h.at[pl.ds(g * OWN + k * CH, CH)])


def _matT(a, w):
    return lax.dot_general(a, w, (((1,), (1,)), ((), ())),
                           preferred_element_type=_f32)


def _tc_prep_body(degp_ref, h0_ref, w_ref, dinv_ref, hs_ref):
    deg = degp_ref[:, 0:1] + 1.0
    dinv = lax.rsqrt(deg)
    dinv_ref[...] = dinv
    hs_ref[...] = _matT(h0_ref[...], w_ref[...]) * dinv


def _layer_tail(p, h, dinv, b, gam, bet):
    out = p * dinv + b
    hn = h + jnp.maximum(out, 0.0)
    mu = jnp.mean(hn, axis=1, keepdims=True)
    d = hn - mu
    var = jnp.mean(d * d, axis=1, keepdims=True)
    return d * lax.rsqrt(var + 1e-5) * gam + bet


def _tc_mid_body(p_ref, h_ref, dinv_ref, b_ref, g_ref, be_ref,
                 w_ref, hn_ref, hs_ref):
    dinv = dinv_ref[...]
    hln = _layer_tail(p_ref[...], h_ref[...], dinv,
                      b_ref[...], g_ref[...], be_ref[...])
    hn_ref[...] = hln
    hs_ref[...] = _matT(hln, w_ref[...]) * dinv


def _tc_fin_body(p_ref, h_ref, dinv_ref, b_ref, g_ref, be_ref,
                 root_ref, wout_ref, out_ref, h3_ref):
    hln = _layer_tail(p_ref[...], h_ref[...], dinv_ref[...],
                      b_ref[...], g_ref[...], be_ref[...])
    h3_ref[...] = hln
    rows = [h3_ref[pl.ds(root_ref[i], 1), :] for i in range(10)]
    rows.append(jnp.zeros((6, H), _f32))
    r = jnp.concatenate(rows, axis=0)
    out_ref[...] = _matT(r, wout_ref[...])


def _tc_prep(degp, h0, w0):
    return pl.pallas_call(
        _tc_prep_body,
        out_shape=[jax.ShapeDtypeStruct((NP, 1), _f32),
                   jax.ShapeDtypeStruct((NP, H), _f32)],
    )(degp, h0, w0)


def _tc_mid(p, h, dinv, b, gam, bet, w_next):
    return pl.pallas_call(
        _tc_mid_body,
        out_shape=[jax.ShapeDtypeStruct((NP, H), _f32),
                   jax.ShapeDtypeStruct((NP, H), _f32)],
    )(p, h, dinv, b, gam, bet, w_next)


def _tc_fin(p, h, dinv, b, gam, bet, root16, w_out):
    vm = pl.BlockSpec(memory_space=pltpu.VMEM)
    return pl.pallas_call(
        _tc_fin_body,
        out_shape=jax.ShapeDtypeStruct((16, 64), _f32),
        in_specs=[vm] * 6 + [pl.BlockSpec(memory_space=pltpu.SMEM), vm],
        scratch_shapes=[pltpu.VMEM((NP, H), _f32)],
    )(p, h, dinv, b, gam, bet, root16, w_out)


def kernel(x, edge_index, batch, cls_emb, nbr_emb, Ws, bs, gammas, betas,
           W_out):
    E = edge_index.shape[1]
    EPS = E + NTILES * CH + SUP

    src_e = edge_index[0].astype(jnp.int32)
    dst_e = edge_index[1].astype(jnp.int32)

    own = dst_e // OWN
    onehot = (own[None, :] == jnp.arange(NTILES, dtype=jnp.int32)[:, None]
              ).astype(jnp.int32)
    cum = jnp.cumsum(onehot, axis=1)
    cnt = cum[:, -1]
    pad = ((cnt + CH - 1) // CH) * CH
    offs = jnp.concatenate([jnp.zeros((1,), jnp.int32),
                            jnp.cumsum(pad, dtype=jnp.int32)[:-1]])
    rank = cum[own, jnp.arange(E, dtype=jnp.int32)] - 1
    slot = offs[own] + rank
    packed = jnp.full((EPS,), OWN, jnp.int32).at[slot].set(
        (src_e << 9) | (dst_e % OWN))
    offs2d = (jnp.zeros((NTILES, 16), jnp.int32)
              .at[:, 0].set(offs).at[:, 1].set(pad // CH))

    xa = jnp.pad(x[:, 0].astype(jnp.int32), (0, NP - N))
    xb = jnp.pad(x[:, 1].astype(jnp.int32), (0, NP - N))
    roots = jnp.searchsorted(batch.astype(jnp.int32),
                             jnp.arange(10, dtype=jnp.int32)).astype(jnp.int32)
    root16 = jnp.pad(roots, (0, 6))

    h0, degp = _sc_embed_deg(xa, xb, cls_emb.astype(_f32),
                             nbr_emb.astype(_f32), packed, offs2d)
    dinv, hs = _tc_prep(degp, h0, Ws[0])

    h = h0
    out16 = None
    for l in range(Ws.shape[0]):
        p = _sc_agg(hs, packed, offs2d)
        b = bs[l].reshape(1, H)
        gam = gammas[l].reshape(1, H)
        bet = betas[l].reshape(1, H)
        if l + 1 < Ws.shape[0]:
            h, hs = _tc_mid(p, h, dinv, b, gam, bet, Ws[l + 1])
        else:
            out16 = _tc_fin(p, h, dinv, b, gam, bet, root16, W_out)
    return out16[:10]


# speedup vs baseline: 2.3676x; 1.0001x over previous
import functools

import jax
import jax.numpy as jnp
from jax import lax
from jax.experimental import pallas as pl
from jax.experimental.pallas import tpu as pltpu
from jax.experimental.pallas import tpu_sc as plsc

N = 10000
NP = 12288
H = 128
CH = 128
NTILES = 32
OWN = NP // NTILES
ACC_ROWS = OWN + 8
_f32 = jnp.float32

_mesh = plsc.VectorSubcoreMesh(core_axis_name="c", subcore_axis_name="s")
SUP = 8192
SUPCH = SUP // CH


@functools.partial(
    pl.kernel,
    out_type=[
        jax.ShapeDtypeStruct((NP, H), _f32),
        jax.ShapeDtypeStruct((NP, 16), _f32),
    ],
    mesh=_mesh,
    scratch_types=[
        pltpu.VMEM((CH,), jnp.int32),
        pltpu.VMEM((CH,), jnp.int32),
        pltpu.VMEM((CH, H), _f32),
        pltpu.VMEM((CH, H), _f32),
        pltpu.VMEM((ACC_ROWS, 16), _f32),
        pltpu.VMEM((CH + 16,), jnp.int32),
        pltpu.VMEM((NTILES, 16), jnp.int32),
        pltpu.SemaphoreType.DMA,
    ],
)
def _sc_embed_deg(xa_h, xb_h, cls_h, nbr_h, pidx_h, offs_h,
                  h0_h, degp_h,
                  idxa_v, idxb_v, rows_v, rows2_v, dacc, exidx_v,
                  offs_v, sem):
    c = lax.axis_index("c")
    s = lax.axis_index("s")
    g = c * 16 + s

    for k in range(OWN // CH):
        base = g * OWN + k * CH
        pltpu.sync_copy(xa_h.at[pl.ds(base, CH)], idxa_v)
        pltpu.sync_copy(xb_h.at[pl.ds(base, CH)], idxb_v)
        pltpu.async_copy(cls_h.at[idxa_v], rows_v, sem).wait()
        pltpu.async_copy(nbr_h.at[idxb_v], rows2_v, sem).wait()

        def addrow(r, carry):
            for j in range(H // 16):
                sl = pl.ds(j * 16, 16)
                rows_v[r, sl] = rows_v[r, sl] + rows2_v[r, sl]
            return carry

        lax.fori_loop(0, CH, addrow, 0)
        pltpu.sync_copy(rows_v, h0_h.at[pl.ds(base, CH)])

    def zrow(r, carry):
        dacc[r, pl.ds(0, 16)] = jnp.zeros((16,), _f32)
        return carry

    lax.fori_loop(0, ACC_ROWS, zrow, 0)
    pltpu.sync_copy(offs_h, offs_v)
    ovec = offs_v[g, pl.ds(0, 16)]
    o_g = pl.multiple_of(ovec[0], CH)
    n_g = ovec[1]

    def dchunk(i, carry):
        off = o_g + i * CH
        pltpu.sync_copy(pidx_h.at[pl.ds(off, CH)],
                        exidx_v.at[pl.ds(0, CH)])

        def dedge(q, carry2):
            base = pl.multiple_of(q * 16, 16)
            rvec = exidx_v[pl.ds(base, 16)] & 511
            for l in range(16):
                r = rvec[l]
                dacc[r, pl.ds(0, 16)] = dacc[r, pl.ds(0, 16)] + 1.0
            return carry2

        lax.fori_loop(0, CH // 16, dedge, 0)
        return carry

    lax.fori_loop(0, n_g, dchunk, 0)

    for k in range(OWN // CH):
        pltpu.sync_copy(dacc.at[pl.ds(k * CH, CH)],
                        degp_h.at[pl.ds(g * OWN + k * CH, CH)])


@functools.partial(
    pl.kernel,
    out_type=jax.ShapeDtypeStruct((NP, H), _f32),
    mesh=_mesh,
    scratch_types=[
        pltpu.VMEM((SUP,), jnp.int32),
        pltpu.VMEM((CH,), jnp.int32),
        pltpu.VMEM((CH,), jnp.int32),
        pltpu.VMEM((CH, H), _f32),
        pltpu.VMEM((CH, H), _f32),
        pltpu.VMEM((ACC_ROWS, H), _f32),
        pltpu.VMEM((NTILES, 16), jnp.int32),
        pltpu.SemaphoreType.DMA,
        pltpu.SemaphoreType.DMA,
    ],
)
def _sc_agg(hs_h, pidx_h, offs_h,
            p_h,
            bigidx_v, src_a, src_b, rows_a, rows_b, acc, offs_v,
            sem_a, sem_b):
    c = lax.axis_index("c")
    s = lax.axis_index("s")
    g = c * 16 + s

    for k in range(OWN // CH):
        pltpu.sync_copy(hs_h.at[pl.ds(g * OWN + k * CH, CH)],
                        acc.at[pl.ds(k * CH, CH)])
    pltpu.sync_copy(offs_h, offs_v)
    ovec = offs_v[g, pl.ds(0, 16)]
    o_g = pl.multiple_of(ovec[0], CH)
    n_g = ovec[1]
    n_sup = (n_g + SUPCH - 1) // SUPCH

    def _unpack(i, sbuf):
        cb = i * CH
        for q in range(CH // 16):
            v = bigidx_v[pl.ds(cb + q * 16, 16)]
            sbuf[pl.ds(q * 16, 16)] = lax.shift_right_logical(v, 9)

    def _accum(i, rbuf):
        cb = i * CH
        for q in range(CH // 16):
            rvec = bigidx_v[pl.ds(cb + q * 16, 16)] & 511
            for l in range(16):
                r = rvec[l]
                e = q * 16 + l
                for j in range(H // 16):
                    sl = pl.ds(j * 16, 16)
                    plsc.addupdate(acc.at[r, sl], rbuf[e, sl])

    def sup_body(sb, carry):
        off = pl.multiple_of(o_g + sb * SUP, CH)
        pltpu.sync_copy(pidx_h.at[pl.ds(off, SUP)], bigidx_v)
        nch = jnp.minimum(n_g - sb * SUPCH, SUPCH)

        _unpack(0, src_a)
        pltpu.async_copy(hs_h.at[src_a], rows_a, sem_a)

        def pair(p, carry2):
            i0 = 2 * p
            i1 = i0 + 1

            @pl.when(i1 < nch)
            def _():
                _unpack(i1, src_b)
                pltpu.async_copy(hs_h.at[src_b], rows_b, sem_b)

            pltpu.make_async_copy(hs_h.at[src_a], rows_a, sem_a).wait()
            _accum(i0, rows_a)

            @pl.when(i0 + 2 < nch)
            def _():
                _unpack(i0 + 2, src_a)
                pltpu.async_copy(hs_h.at[src_a], rows_a, sem_a)

            @pl.when(i1 < nch)
            def _():
                pltpu.make_async_copy(hs_h.at[src_b], rows_b, sem_b).wait()
                _accum(i1, rows_b)

            return carry2

        lax.fori_loop(0, (nch + 1) // 2, pair, 0)
        return carry

    lax.fori_loop(0, n_sup, sup_body, 0)

    for k in range(OWN // CH):
        pltpu.sync_copy(acc.at[pl.ds(k * CH, CH)],
                        p_h.at[pl.ds(g * OWN + k * CH, CH)])


def _matT(a, w):
    return lax.dot_general(a, w, (((1,), (1,)), ((), ())),
                           preferred_element_type=_f32)


def _tc_prep_body(degp_ref, h0_ref, w_ref, dinv_ref, hs_ref):
    deg = degp_ref[:, 0:1] + 1.0
    dinv = lax.rsqrt(deg)
    dinv_ref[...] = dinv
    hs_ref[...] = _matT(h0_ref[...], w_ref[...]) * dinv


def _layer_tail(p, h, dinv, b, gam, bet):
    out = p * dinv + b
    hn = h + jnp.maximum(out, 0.0)
    mu = jnp.mean(hn, axis=1, keepdims=True)
    d = hn - mu
    var = jnp.mean(d * d, axis=1, keepdims=True)
    return d * lax.rsqrt(var + 1e-5) * gam + bet


def _tc_mid_body(p_ref, h_ref, dinv_ref, b_ref, g_ref, be_ref,
                 w_ref, hn_ref, hs_ref):
    dinv = dinv_ref[...]
    hln = _layer_tail(p_ref[...], h_ref[...], dinv,
                      b_ref[...], g_ref[...], be_ref[...])
    hn_ref[...] = hln
    hs_ref[...] = _matT(hln, w_ref[...]) * dinv


def _tc_fin_body(p_ref, h_ref, dinv_ref, b_ref, g_ref, be_ref,
                 root_ref, wout_ref, out_ref, h3_ref):
    hln = _layer_tail(p_ref[...], h_ref[...], dinv_ref[...],
                      b_ref[...], g_ref[...], be_ref[...])
    h3_ref[...] = hln
    rows = [h3_ref[pl.ds(root_ref[i], 1), :] for i in range(10)]
    rows.append(jnp.zeros((6, H), _f32))
    r = jnp.concatenate(rows, axis=0)
    out_ref[...] = _matT(r, wout_ref[...])


def _tc_prep(degp, h0, w0):
    return pl.pallas_call(
        _tc_prep_body,
        out_shape=[jax.ShapeDtypeStruct((NP, 1), _f32),
                   jax.ShapeDtypeStruct((NP, H), _f32)],
    )(degp, h0, w0)


def _tc_mid(p, h, dinv, b, gam, bet, w_next):
    return pl.pallas_call(
        _tc_mid_body,
        out_shape=[jax.ShapeDtypeStruct((NP, H), _f32),
                   jax.ShapeDtypeStruct((NP, H), _f32)],
    )(p, h, dinv, b, gam, bet, w_next)


def _tc_fin(p, h, dinv, b, gam, bet, root16, w_out):
    vm = pl.BlockSpec(memory_space=pltpu.VMEM)
    return pl.pallas_call(
        _tc_fin_body,
        out_shape=jax.ShapeDtypeStruct((16, 64), _f32),
        in_specs=[vm] * 6 + [pl.BlockSpec(memory_space=pltpu.SMEM), vm],
        scratch_shapes=[pltpu.VMEM((NP, H), _f32)],
    )(p, h, dinv, b, gam, bet, root16, w_out)


def kernel(x, edge_index, batch, cls_emb, nbr_emb, Ws, bs, gammas, betas,
           W_out):
    E = edge_index.shape[1]
    EPS = E + NTILES * CH + SUP

    src_e = edge_index[0].astype(jnp.int32)
    dst_e = edge_index[1].astype(jnp.int32)

    own = dst_e // OWN
    onehot = (own[None, :] == jnp.arange(NTILES, dtype=jnp.int32)[:, None]
              ).astype(jnp.int32)
    cum = jnp.cumsum(onehot, axis=1)
    cnt = cum[:, -1]
    pad = ((cnt + CH - 1) // CH) * CH
    offs = jnp.concatenate([jnp.zeros((1,), jnp.int32),
                            jnp.cumsum(pad, dtype=jnp.int32)[:-1]])
    rank = cum[own, jnp.arange(E, dtype=jnp.int32)] - 1
    slot = offs[own] + rank
    packed = jnp.full((EPS,), OWN, jnp.int32).at[slot].set(
        (src_e << 9) | (dst_e % OWN))
    offs2d = (jnp.zeros((NTILES, 16), jnp.int32)
              .at[:, 0].set(offs).at[:, 1].set(pad // CH))

    xa = jnp.pad(x[:, 0].astype(jnp.int32), (0, NP - N))
    xb = jnp.pad(x[:, 1].astype(jnp.int32), (0, NP - N))
    roots = jnp.searchsorted(batch.astype(jnp.int32),
                             jnp.arange(10, dtype=jnp.int32)).astype(jnp.int32)
    root16 = jnp.pad(roots, (0, 6))

    h0, degp = _sc_embed_deg(xa, xb, cls_emb.astype(_f32),
                             nbr_emb.astype(_f32), packed, offs2d)
    dinv, hs = _tc_prep(degp, h0, Ws[0])

    h = h0
    out16 = None
    for l in range(Ws.shape[0]):
        p = _sc_agg(hs, packed, offs2d)
        b = bs[l].reshape(1, H)
        gam = gammas[l].reshape(1, H)
        bet = betas[l].reshape(1, H)
        if l + 1 < Ws.shape[0]:
            h, hs = _tc_mid(p, h, dinv, b, gam, bet, Ws[l + 1])
        else:
            out16 = _tc_fin(p, h, dinv, b, gam, bet, root16, W_out)
    return out16[:10]


# speedup vs baseline: 2.4975x; 1.0549x over previous
import functools

import jax
import jax.numpy as jnp
from jax import lax
from jax.experimental import pallas as pl
from jax.experimental.pallas import tpu as pltpu
from jax.experimental.pallas import tpu_sc as plsc

N = 10000
NP = 12288
H = 128
CH = 128
NTILES = 32
OWN = NP // NTILES
ACC_ROWS = OWN + 8
_f32 = jnp.float32

_mesh = plsc.VectorSubcoreMesh(core_axis_name="c", subcore_axis_name="s")
SUP = 8192
SUPCH = SUP // CH


@functools.partial(
    pl.kernel,
    out_type=[
        jax.ShapeDtypeStruct((NP, H), _f32),
        jax.ShapeDtypeStruct((NP, 16), _f32),
    ],
    mesh=_mesh,
    scratch_types=[
        pltpu.VMEM((CH,), jnp.int32),
        pltpu.VMEM((CH,), jnp.int32),
        pltpu.VMEM((CH, H), _f32),
        pltpu.VMEM((CH, H), _f32),
        pltpu.VMEM((ACC_ROWS, 16), _f32),
        pltpu.VMEM((CH + 16,), jnp.int32),
        pltpu.VMEM((NTILES, 16), jnp.int32),
        pltpu.SemaphoreType.DMA,
    ],
)
def _sc_embed_deg(xa_h, xb_h, cls_h, nbr_h, pidx_h, offs_h,
                  h0_h, degp_h,
                  idxa_v, idxb_v, rows_v, rows2_v, dacc, exidx_v,
                  offs_v, sem):
    c = lax.axis_index("c")
    s = lax.axis_index("s")
    g = c * 16 + s

    for k in range(OWN // CH):
        base = g * OWN + k * CH
        pltpu.sync_copy(xa_h.at[pl.ds(base, CH)], idxa_v)
        pltpu.sync_copy(xb_h.at[pl.ds(base, CH)], idxb_v)
        pltpu.async_copy(cls_h.at[idxa_v], rows_v, sem).wait()
        pltpu.async_copy(nbr_h.at[idxb_v], rows2_v, sem).wait()

        def addrow(r, carry):
            for j in range(H // 16):
                sl = pl.ds(j * 16, 16)
                rows_v[r, sl] = rows_v[r, sl] + rows2_v[r, sl]
            return carry

        lax.fori_loop(0, CH, addrow, 0)
        pltpu.sync_copy(rows_v, h0_h.at[pl.ds(base, CH)])

    def zrow(r, carry):
        dacc[r, pl.ds(0, 16)] = jnp.zeros((16,), _f32)
        return carry

    lax.fori_loop(0, ACC_ROWS, zrow, 0)
    pltpu.sync_copy(offs_h, offs_v)
    ovec = offs_v[g, pl.ds(0, 16)]
    o_g = pl.multiple_of(ovec[0], CH)
    n_g = ovec[1]

    def dchunk(i, carry):
        off = o_g + i * CH
        pltpu.sync_copy(pidx_h.at[pl.ds(off, CH)],
                        exidx_v.at[pl.ds(0, CH)])

        def dedge(q, carry2):
            base = pl.multiple_of(q * 16, 16)
            rvec = exidx_v[pl.ds(base, 16)] & 511
            for l in range(16):
                r = rvec[l]
                dacc[r, pl.ds(0, 16)] = dacc[r, pl.ds(0, 16)] + 1.0
            return carry2

        lax.fori_loop(0, CH // 16, dedge, 0)
        return carry

    lax.fori_loop(0, n_g, dchunk, 0)

    for k in range(OWN // CH):
        pltpu.sync_copy(dacc.at[pl.ds(k * CH, CH)],
                        degp_h.at[pl.ds(g * OWN + k * CH, CH)])


@functools.partial(
    pl.kernel,
    out_type=jax.ShapeDtypeStruct((NP, H), _f32),
    mesh=_mesh,
    scratch_types=[
        pltpu.VMEM((SUP,), jnp.int32),
        pltpu.VMEM((CH,), jnp.int32),
        pltpu.VMEM((CH,), jnp.int32),
        pltpu.VMEM((CH, H), _f32),
        pltpu.VMEM((CH, H), _f32),
        pltpu.VMEM((ACC_ROWS, H), _f32),
        pltpu.VMEM((NTILES, 16), jnp.int32),
        pltpu.SemaphoreType.DMA,
        pltpu.SemaphoreType.DMA,
    ],
)
def _sc_agg(hs_h, pidx_h, offs_h,
            p_h,
            bigidx_v, src_a, src_b, rows_a, rows_b, acc, offs_v,
            sem_a, sem_b):
    c = lax.axis_index("c")
    s = lax.axis_index("s")
    g = c * 16 + s

    for k in range(OWN // CH):
        pltpu.sync_copy(hs_h.at[pl.ds(g * OWN + k * CH, CH)],
                        acc.at[pl.ds(k * CH, CH)])
    pltpu.sync_copy(offs_h, offs_v)
    ovec = offs_v[g, pl.ds(0, 16)]
    o_g = pl.multiple_of(ovec[0], CH)
    n_g = ovec[1]
    n_sup = (n_g + SUPCH - 1) // SUPCH

    def _unpack(i, sbuf):
        cb = i * CH
        for q in range(CH // 16):
            v = bigidx_v[pl.ds(cb + q * 16, 16)]
            sbuf[pl.ds(q * 16, 16)] = lax.shift_right_logical(v, 9)

    def _accum(i, rbuf):
        cb = i * CH
        for q in range(CH // 16):
            rvec = bigidx_v[pl.ds(cb + q * 16, 16)] & 511
            for l in range(16):
                r = rvec[l]
                e = q * 16 + l
                for j in range(H // 16):
                    sl = pl.ds(j * 16, 16)
                    plsc.addupdate(acc.at[r, sl], rbuf[e, sl])

    def sup_body(sb, carry):
        off = pl.multiple_of(o_g + sb * SUP, CH)
        pltpu.sync_copy(pidx_h.at[pl.ds(off, SUP)], bigidx_v)
        nch = jnp.minimum(n_g - sb * SUPCH, SUPCH)

        _unpack(0, src_a)
        pltpu.async_copy(hs_h.at[src_a], rows_a, sem_a)

        def pair(p, carry2):
            i0 = 2 * p
            i1 = i0 + 1

            @pl.when(i1 < nch)
            def _():
                _unpack(i1, src_b)
                pltpu.async_copy(hs_h.at[src_b], rows_b, sem_b)

            pltpu.make_async_copy(hs_h.at[src_a], rows_a, sem_a).wait()
            _accum(i0, rows_a)

            @pl.when(i0 + 2 < nch)
            def _():
                _unpack(i0 + 2, src_a)
                pltpu.async_copy(hs_h.at[src_a], rows_a, sem_a)

            @pl.when(i1 < nch)
            def _():
                pltpu.make_async_copy(hs_h.at[src_b], rows_b, sem_b).wait()
                _accum(i1, rows_b)

            return carry2

        lax.fori_loop(0, (nch + 1) // 2, pair, 0)
        return carry

    lax.fori_loop(0, n_sup, sup_body, 0)

    for k in range(OWN // CH):
        pltpu.sync_copy(acc.at[pl.ds(k * CH, CH)],
                        p_h.at[pl.ds(g * OWN + k * CH, CH)])


def _matT(a, w):
    return lax.dot_general(a, w, (((1,), (1,)), ((), ())),
                           preferred_element_type=_f32)


def _tc_prep_body(degp_ref, h0_ref, w_ref, dinv_ref, hs_ref):
    deg = degp_ref[:, 0:1] + 1.0
    dinv = lax.rsqrt(deg)
    dinv_ref[...] = dinv
    hs_ref[...] = _matT(h0_ref[...], w_ref[...]) * dinv


def _layer_tail(p, h, dinv, b, gam, bet):
    out = p * dinv + b
    hn = h + jnp.maximum(out, 0.0)
    mu = jnp.mean(hn, axis=1, keepdims=True)
    d = hn - mu
    var = jnp.mean(d * d, axis=1, keepdims=True)
    return d * lax.rsqrt(var + 1e-5) * gam + bet


def _tc_mid_body(p_ref, h_ref, dinv_ref, b_ref, g_ref, be_ref,
                 w_ref, hn_ref, hs_ref):
    dinv = dinv_ref[...]
    hln = _layer_tail(p_ref[...], h_ref[...], dinv,
                      b_ref[...], g_ref[...], be_ref[...])
    hn_ref[...] = hln
    hs_ref[...] = _matT(hln, w_ref[...]) * dinv


def _tc_fin_body(p_ref, h_ref, dinv_ref, b_ref, g_ref, be_ref,
                 root_ref, wout_ref, out_ref, h3_ref):
    hln = _layer_tail(p_ref[...], h_ref[...], dinv_ref[...],
                      b_ref[...], g_ref[...], be_ref[...])
    h3_ref[...] = hln
    rows = [h3_ref[pl.ds(root_ref[i], 1), :] for i in range(10)]
    rows.append(jnp.zeros((6, H), _f32))
    r = jnp.concatenate(rows, axis=0)
    out_ref[...] = _matT(r, wout_ref[...])


def _tc_prep(degp, h0, w0):
    return pl.pallas_call(
        _tc_prep_body,
        out_shape=[jax.ShapeDtypeStruct((NP, 1), _f32),
                   jax.ShapeDtypeStruct((NP, H), _f32)],
    )(degp, h0, w0)


def _tc_mid(p, h, dinv, b, gam, bet, w_next):
    return pl.pallas_call(
        _tc_mid_body,
        out_shape=[jax.ShapeDtypeStruct((NP, H), _f32),
                   jax.ShapeDtypeStruct((NP, H), _f32)],
    )(p, h, dinv, b, gam, bet, w_next)


def _tc_fin(p, h, dinv, b, gam, bet, root16, w_out):
    vm = pl.BlockSpec(memory_space=pltpu.VMEM)
    return pl.pallas_call(
        _tc_fin_body,
        out_shape=jax.ShapeDtypeStruct((16, 64), _f32),
        in_specs=[vm] * 6 + [pl.BlockSpec(memory_space=pltpu.SMEM), vm],
        scratch_shapes=[pltpu.VMEM((NP, H), _f32)],
    )(p, h, dinv, b, gam, bet, root16, w_out)


def kernel(x, edge_index, batch, cls_emb, nbr_emb, Ws, bs, gammas, betas,
           W_out):
    E = edge_index.shape[1]
    EPS = E + NTILES * CH + SUP

    src_e = edge_index[0].astype(jnp.int32)
    dst_e = edge_index[1].astype(jnp.int32)

    own = dst_e // OWN
    onehot = (own[None, :] == jnp.arange(NTILES, dtype=jnp.int32)[:, None]
              ).astype(jnp.int32)
    cum = jnp.cumsum(onehot, axis=1)
    cnt = cum[:, -1]
    pad = ((cnt + CH - 1) // CH) * CH
    offs = jnp.concatenate([jnp.zeros((1,), jnp.int32),
                            jnp.cumsum(pad, dtype=jnp.int32)[:-1]])
    slot = jnp.sum((cum + (offs - 1)[:, None]) * onehot, axis=0)
    packed = jnp.full((EPS,), OWN, jnp.int32).at[slot].set(
        (src_e << 9) | (dst_e % OWN))
    offs2d = (jnp.zeros((NTILES, 16), jnp.int32)
              .at[:, 0].set(offs).at[:, 1].set(pad // CH))

    xa = jnp.pad(x[:, 0].astype(jnp.int32), (0, NP - N))
    xb = jnp.pad(x[:, 1].astype(jnp.int32), (0, NP - N))
    roots = jnp.searchsorted(batch.astype(jnp.int32),
                             jnp.arange(10, dtype=jnp.int32)).astype(jnp.int32)
    root16 = jnp.pad(roots, (0, 6))

    h0, degp = _sc_embed_deg(xa, xb, cls_emb.astype(_f32),
                             nbr_emb.astype(_f32), packed, offs2d)
    dinv, hs = _tc_prep(degp, h0, Ws[0])

    h = h0
    out16 = None
    for l in range(Ws.shape[0]):
        p = _sc_agg(hs, packed, offs2d)
        b = bs[l].reshape(1, H)
        gam = gammas[l].reshape(1, H)
        bet = betas[l].reshape(1, H)
        if l + 1 < Ws.shape[0]:
            h, hs = _tc_mid(p, h, dinv, b, gam, bet, Ws[l + 1])
        else:
            out16 = _tc_fin(p, h, dinv, b, gam, bet, root16, W_out)
    return out16[:10]
